# baseline (speedup 1.0000x reference)
import functools

import jax
import jax.numpy as jnp
from jax import lax
from jax.experimental import pallas as pl
from jax.experimental.pallas import tpu as pltpu
from jax.experimental.pallas import tpu_sc as plsc

N = 10000
E = 320000
B = 64
NP = N + 16
CHUNK = 10240
RB = 10
RBS = N // RB


def _tc_head1(x, W, a_s, a_d, dout):
    def body(x_ref, w_ref, as_ref, ad_ref, h_ref, aso_ref, ado_ref):
        h = jnp.dot(x_ref[...], w_ref[...], preferred_element_type=jnp.float32)
        h_ref[...] = h.astype(jnp.bfloat16)
        asv = lax.dot_general(as_ref[...], h, (((1,), (1,)), ((), ())),
                              preferred_element_type=jnp.float32)
        adv = lax.dot_general(ad_ref[...], h, (((1,), (1,)), ((), ())),
                              preferred_element_type=jnp.float32)
        aso_ref[...] = asv.reshape(1, 1, RBS)
        ado_ref[...] = adv.reshape(1, 1, RBS)

    din = x.shape[1]
    h, as3, ad3 = pl.pallas_call(
        body,
        grid=(RB,),
        in_specs=[
            pl.BlockSpec((RBS, din), lambda i: (i, 0)),
            pl.BlockSpec((din, dout), lambda i: (0, 0)),
            pl.BlockSpec((1, dout), lambda i: (0, 0)),
            pl.BlockSpec((1, dout), lambda i: (0, 0)),
        ],
        out_specs=[
            pl.BlockSpec((RBS, dout), lambda i: (i, 0)),
            pl.BlockSpec((1, 1, RBS), lambda i: (i, 0, 0)),
            pl.BlockSpec((1, 1, RBS), lambda i: (i, 0, 0)),
        ],
        out_shape=[
            jax.ShapeDtypeStruct((N, dout), jnp.bfloat16),
            jax.ShapeDtypeStruct((RB, 1, RBS), jnp.float32),
            jax.ShapeDtypeStruct((RB, 1, RBS), jnp.float32),
        ],
    )(x, W, a_s.reshape(1, dout), a_d.reshape(1, dout))
    return h, as3.reshape(N), ad3.reshape(N)


def _tc_head2(outp, b, W, a_s, a_d, dout):
    def body(op_ref, b_ref, w_ref, as_ref, ad_ref, h_ref, aso_ref, ado_ref):
        x = jax.nn.relu(op_ref[0].astype(jnp.float32) +
                        op_ref[1].astype(jnp.float32) + b_ref[...])
        h = jnp.dot(x, w_ref[...], preferred_element_type=jnp.float32)
        h_ref[...] = h.astype(jnp.bfloat16)
        asv = lax.dot_general(as_ref[...], h, (((1,), (1,)), ((), ())),
                              preferred_element_type=jnp.float32)
        adv = lax.dot_general(ad_ref[...], h, (((1,), (1,)), ((), ())),
                              preferred_element_type=jnp.float32)
        aso_ref[...] = asv.reshape(1, 1, RBS)
        ado_ref[...] = adv.reshape(1, 1, RBS)

    din = outp.shape[2]
    h, as3, ad3 = pl.pallas_call(
        body,
        grid=(RB,),
        in_specs=[
            pl.BlockSpec((2, RBS, din), lambda i: (0, i, 0)),
            pl.BlockSpec((1, din), lambda i: (0, 0)),
            pl.BlockSpec((din, dout), lambda i: (0, 0)),
            pl.BlockSpec((1, dout), lambda i: (0, 0)),
            pl.BlockSpec((1, dout), lambda i: (0, 0)),
        ],
        out_specs=[
            pl.BlockSpec((RBS, dout), lambda i: (i, 0)),
            pl.BlockSpec((1, 1, RBS), lambda i: (i, 0, 0)),
            pl.BlockSpec((1, 1, RBS), lambda i: (i, 0, 0)),
        ],
        out_shape=[
            jax.ShapeDtypeStruct((N, dout), jnp.bfloat16),
            jax.ShapeDtypeStruct((RB, 1, RBS), jnp.float32),
            jax.ShapeDtypeStruct((RB, 1, RBS), jnp.float32),
        ],
    )(outp, b.reshape(1, din), W, a_s.reshape(1, dout), a_d.reshape(1, dout))
    return h, as3.reshape(N), ad3.reshape(N)


_SC_PARAMS = pltpu.CompilerParams(needs_layout_passes=False,
                                  use_tc_tiling_on_sc=False)
_MESH = plsc.VectorSubcoreMesh(core_axis_name="c", subcore_axis_name="s")


def _sc_gat_den(asrc, adst, src2d, dst2d):
    @functools.partial(
        pl.kernel,
        out_type=[
            jax.ShapeDtypeStruct((32 * CHUNK // 128, 128), jnp.float32),
            jax.ShapeDtypeStruct((2, NP), jnp.float32),
        ],
        mesh=_MESH,
        compiler_params=_SC_PARAMS,
        scratch_types=[
            pltpu.VMEM((NP,), jnp.float32),
            pltpu.VMEM((NP,), jnp.float32),
            pltpu.VMEM((80, 128), jnp.int32),
            pltpu.VMEM((80, 128), jnp.int32),
            pltpu.VMEM((80, 128), jnp.float32),
            pltpu.VMEM((640,), jnp.float32),
            pltpu.VMEM_SHARED((NP,), jnp.float32),
            pltpu.SemaphoreType.DMA,
            pltpu.SemaphoreType.DMA,
        ],
    )
    def k(asrc_hbm, adst_hbm, src_hbm, dst_hbm, ex_hbm, den_hbm,
          asrc_loc, adst_loc, src_loc, dst_loc, ex_loc, zeros, den_sp,
          lsem, dsem):
        cid = lax.axis_index("c")
        sid = lax.axis_index("s")
        chunk = cid * 16 + sid
        z16 = jnp.zeros((16,), jnp.float32)

        pltpu.async_copy(asrc_hbm, asrc_loc.at[pl.ds(0, N)], lsem)
        pltpu.async_copy(adst_hbm, adst_loc.at[pl.ds(0, N)], lsem)
        pltpu.async_copy(src_hbm.at[pl.ds(chunk * 80, 80)], src_loc, lsem)
        pltpu.async_copy(dst_hbm.at[pl.ds(chunk * 80, 80)], dst_loc, lsem)

        @pl.loop(0, 640, step=16)
        def _(i):
            zeros[pl.ds(i, 16)] = z16

        @pl.when(sid == 0)
        def _():
            for t in range(15):
                pltpu.sync_copy(zeros, den_sp.at[pl.ds(t * 640, 640)])
            pltpu.sync_copy(zeros.at[pl.ds(0, 416)], den_sp.at[pl.ds(9600, 416)])

        pltpu.make_async_copy(asrc_hbm, asrc_loc.at[pl.ds(0, N)], lsem).wait()
        pltpu.make_async_copy(adst_hbm, adst_loc.at[pl.ds(0, N)], lsem).wait()
        pltpu.make_async_copy(src_hbm.at[pl.ds(chunk * 80, 80)], src_loc, lsem).wait()
        pltpu.make_async_copy(dst_hbm.at[pl.ds(chunk * 80, 80)], dst_loc, lsem).wait()
        asrc_loc[pl.ds(N, 16)] = z16
        adst_loc[pl.ds(N, 16)] = z16
        plsc.subcore_barrier()

        @pl.loop(0, 80)
        def _(jb):
            for g in range(8):
                sv = src_loc[jb, pl.ds(16 * g, 16)]
                dv = dst_loc[jb, pl.ds(16 * g, 16)]
                e = plsc.load_gather(asrc_loc, [sv]) + plsc.load_gather(adst_loc, [dv])
                e = jnp.where(e > 0, e, 0.2 * e)
                ex_loc[jb, pl.ds(16 * g, 16)] = jnp.exp(e)

        pltpu.async_copy(ex_loc, ex_hbm.at[pl.ds(chunk * 80, 80)], lsem)

        for b in range(5):
            @pl.loop(16 * b, 16 * (b + 1))
            def _(jb):
                pltpu.async_copy(ex_loc.at[jb], den_sp.at[dst_loc.at[jb]],
                                 dsem, add=True)

            @pl.loop(16 * b, 16 * (b + 1))
            def _(jb):
                pltpu.make_async_copy(ex_loc.at[jb], den_sp.at[dst_loc.at[jb]],
                                      dsem).wait()

        pltpu.make_async_copy(ex_loc, ex_hbm.at[pl.ds(chunk * 80, 80)], lsem).wait()
        plsc.subcore_barrier()

        @pl.when(sid == 0)
        def _():
            pltpu.sync_copy(den_sp, den_hbm.at[cid])

    return k(asrc, adst, src2d, dst2d)


def _sc_gat_agg(h, ex, den, src2d, dst2d, dout):
    @functools.partial(
        pl.kernel,
        out_type=jax.ShapeDtypeStruct((2, N, dout), jnp.bfloat16),
        mesh=_MESH,
        compiler_params=_SC_PARAMS,
        scratch_types=[
            pltpu.VMEM((NP,), jnp.float32),
            pltpu.VMEM((NP,), jnp.float32),
            pltpu.VMEM((80, 128), jnp.int32),
            pltpu.VMEM((80, 128), jnp.int32),
            pltpu.VMEM((80, 128), jnp.float32),
            pltpu.VMEM((256, dout), jnp.bfloat16),
            pltpu.VMEM((256, dout), jnp.bfloat16),
            pltpu.VMEM((256, dout), jnp.bfloat16),
            pltpu.VMEM((256, dout), jnp.bfloat16),
            pltpu.VMEM((256,), jnp.float32),
            pltpu.VMEM_SHARED((NP, dout), jnp.bfloat16),
            pltpu.SemaphoreType.DMA,
            pltpu.SemaphoreType.DMA,
            pltpu.SemaphoreType.DMA,
            pltpu.SemaphoreType.DMA,
            pltpu.SemaphoreType.DMA,
        ],
    )
    def k(h_hbm, ex_hbm, den_hbm, src_hbm, dst_hbm, out_hbm,
          den_loc, den1, src_loc, dst_loc, ex_loc, bufa, bufb, sba, sbb,
          alpha, out_sp, lsem, gsa, gsb, ssa, ssb):
        cid = lax.axis_index("c")
        sid = lax.axis_index("s")
        chunk = cid * 16 + sid
        z32b = jnp.zeros((32,), jnp.bfloat16)

        pltpu.async_copy(den_hbm.at[0], den_loc, lsem)
        pltpu.async_copy(den_hbm.at[1], den1, lsem)
        pltpu.async_copy(src_hbm.at[pl.ds(chunk * 80, 80)], src_loc, lsem)
        pltpu.async_copy(dst_hbm.at[pl.ds(chunk * 80, 80)], dst_loc, lsem)
        pltpu.async_copy(ex_hbm.at[pl.ds(chunk * 80, 80)], ex_loc, lsem)

        @pl.loop(0, 256)
        def _(r):
            for q in range(dout // 32):
                sba[r, pl.ds(32 * q, 32)] = z32b

        zbase = sid * 626
        for t in range(2):
            pltpu.sync_copy(sba, out_sp.at[pl.ds(zbase + 256 * t, 256)])
        pltpu.sync_copy(sba.at[pl.ds(0, 114)], out_sp.at[pl.ds(zbase + 512, 114)])

        pltpu.make_async_copy(den_hbm.at[0], den_loc, lsem).wait()
        pltpu.make_async_copy(den_hbm.at[1], den1, lsem).wait()
        pltpu.make_async_copy(src_hbm.at[pl.ds(chunk * 80, 80)], src_loc, lsem).wait()
        pltpu.make_async_copy(dst_hbm.at[pl.ds(chunk * 80, 80)], dst_loc, lsem).wait()
        pltpu.make_async_copy(ex_hbm.at[pl.ds(chunk * 80, 80)], ex_loc, lsem).wait()

        @pl.loop(0, NP, step=16)
        def _(i):
            den_loc[pl.ds(i, 16)] = den_loc[pl.ds(i, 16)] + den1[pl.ds(i, 16)]

        plsc.subcore_barrier()

        def fire_gather(j, buf, t, sem):
            pltpu.async_copy(h_hbm.at[src_loc.at[j]],
                             buf.at[pl.ds(128 * t, 128)], sem)

        def wait_gather(j, buf, t, sem):
            pltpu.make_async_copy(h_hbm.at[src_loc.at[j]],
                                  buf.at[pl.ds(128 * t, 128)], sem).wait()

        def fire_scatter(j, sbuf, t, sem):
            pltpu.async_copy(sbuf.at[pl.ds(128 * t, 128)],
                             out_sp.at[dst_loc.at[j]], sem, add=True)

        def wait_scatter(j, sbuf, t, sem):
            pltpu.make_async_copy(sbuf.at[pl.ds(128 * t, 128)],
                                  out_sp.at[dst_loc.at[j]], sem).wait()

        def alpha_scale(jj, buf, sbuf):
            for t in range(2):
                for g in range(8):
                    dv = dst_loc[jj + t, pl.ds(16 * g, 16)]
                    dn = plsc.load_gather(den_loc, [dv])
                    exv = ex_loc[jj + t, pl.ds(16 * g, 16)]
                    alpha[pl.ds(128 * t + 16 * g, 16)] = exv / (dn + 1e-16)

            @pl.loop(0, 256, step=16)
            def _(e0):
                av16 = alpha[pl.ds(e0, 16)]
                for k2 in range(16):
                    avf = jnp.broadcast_to(av16[k2], (16,))
                    av = plsc.pack(avf, avf, format=plsc.PackFormat.INTERLEAVED)
                    for q in range(dout // 32):
                        sbuf[e0 + k2, pl.ds(32 * q, 32)] = (
                            buf[e0 + k2, pl.ds(32 * q, 32)] * av)

        fire_gather(0, bufa, 0, gsa)
        fire_gather(1, bufa, 1, gsa)

        @pl.loop(0, 80, step=4)
        def _(jj):
            fire_gather(jj + 2, bufb, 0, gsb)
            fire_gather(jj + 3, bufb, 1, gsb)
            wait_gather(jj, bufa, 0, gsa)
            wait_gather(jj + 1, bufa, 1, gsa)

            @pl.when(jj > 0)
            def _():
                wait_scatter(jj - 4, sba, 0, ssa)
                wait_scatter(jj - 3, sba, 1, ssa)

            alpha_scale(jj, bufa, sba)
            fire_scatter(jj, sba, 0, ssa)
            fire_scatter(jj + 1, sba, 1, ssa)

            @pl.when(jj < 76)
            def _():
                fire_gather(jj + 4, bufa, 0, gsa)
                fire_gather(jj + 5, bufa, 1, gsa)

            wait_gather(jj + 2, bufb, 0, gsb)
            wait_gather(jj + 3, bufb, 1, gsb)

            @pl.when(jj > 0)
            def _():
                wait_scatter(jj - 2, sbb, 0, ssb)
                wait_scatter(jj - 1, sbb, 1, ssb)

            alpha_scale(jj + 2, bufb, sbb)
            fire_scatter(jj + 2, sbb, 0, ssb)
            fire_scatter(jj + 3, sbb, 1, ssb)

        wait_scatter(76, sba, 0, ssa)
        wait_scatter(77, sba, 1, ssa)
        wait_scatter(78, sbb, 0, ssb)
        wait_scatter(79, sbb, 1, ssb)
        plsc.subcore_barrier()
        wb = sid * 624
        pltpu.sync_copy(out_sp.at[pl.ds(wb, 624)], out_hbm.at[cid, pl.ds(wb, 624)])

        @pl.when(sid == 15)
        def _():
            pltpu.sync_copy(out_sp.at[pl.ds(9984, 16)],
                            out_hbm.at[cid, pl.ds(9984, 16)])

    return k(h, ex, den, src2d, dst2d)


def _sc_gat_edges(h, asrc, adst, src2d, dst2d, dout):
    ex, den = _sc_gat_den(asrc, adst, src2d, dst2d)
    return _sc_gat_agg(h, ex, den, src2d, dst2d, dout)


def _tc_pool_final(op_s, b_s, op_t, b_t, xsb3, xtb3, W_lin, b_lin):
    def body(ops_ref, bs_ref, opt_ref, bt_ref, xsb_ref, xtb_ref, wl_ref, bl_ref,
             out_ref, accs, cnts, acct, cntt):
        i = pl.program_id(0)

        @pl.when(i == 0)
        def _():
            accs[...] = jnp.zeros_like(accs)
            cnts[...] = jnp.zeros_like(cnts)
            acct[...] = jnp.zeros_like(acct)
            cntt[...] = jnp.zeros_like(cntt)

        iot = lax.broadcasted_iota(jnp.int32, (B, RBS), 0)
        x2s = jax.nn.relu(ops_ref[0].astype(jnp.float32) +
                          ops_ref[1].astype(jnp.float32) + bs_ref[...])
        ms = (xsb_ref[0, 0, :][None, :] == iot).astype(jnp.float32)
        accs[...] += jnp.dot(ms, x2s, preferred_element_type=jnp.float32)
        cnts[...] += jnp.sum(ms, axis=1, keepdims=True)
        x2t = jax.nn.relu(opt_ref[0].astype(jnp.float32) +
                          opt_ref[1].astype(jnp.float32) + bt_ref[...])
        mt = (xtb_ref[0, 0, :][None, :] == iot).astype(jnp.float32)
        acct[...] += jnp.dot(mt, x2t, preferred_element_type=jnp.float32)
        cntt[...] += jnp.sum(mt, axis=1, keepdims=True)

        @pl.when(i == RB - 1)
        def _():
            xs = accs[...] / jnp.maximum(cnts[...], 1.0)
            xt = acct[...] / jnp.maximum(cntt[...], 1.0)
            o = jnp.dot(xs + xt, wl_ref[...], preferred_element_type=jnp.float32)
            out_ref[...] = jax.nn.sigmoid(o + bl_ref[...])

    din = op_s.shape[2]
    return pl.pallas_call(
        body,
        grid=(RB,),
        in_specs=[
            pl.BlockSpec((2, RBS, din), lambda i: (0, i, 0)),
            pl.BlockSpec((1, din), lambda i: (0, 0)),
            pl.BlockSpec((2, RBS, din), lambda i: (0, i, 0)),
            pl.BlockSpec((1, din), lambda i: (0, 0)),
            pl.BlockSpec((1, 1, RBS), lambda i: (i, 0, 0)),
            pl.BlockSpec((1, 1, RBS), lambda i: (i, 0, 0)),
            pl.BlockSpec((din, 1), lambda i: (0, 0)),
            pl.BlockSpec((1, 1), lambda i: (0, 0)),
        ],
        out_specs=pl.BlockSpec((B, 1), lambda i: (0, 0)),
        out_shape=jax.ShapeDtypeStruct((B, 1), jnp.float32),
        scratch_shapes=[
            pltpu.VMEM((B, din), jnp.float32),
            pltpu.VMEM((B, 1), jnp.float32),
            pltpu.VMEM((B, din), jnp.float32),
            pltpu.VMEM((B, 1), jnp.float32),
        ],
    )(op_s, b_s.reshape(1, din), op_t, b_t.reshape(1, din),
      xsb3, xtb3, W_lin, b_lin.reshape(1, 1))


def _pad_edges(edge_index):
    src = edge_index[0].reshape(32, E // 32)
    dst = edge_index[1].reshape(32, E // 32)
    src = jnp.pad(src, ((0, 0), (0, CHUNK - E // 32)), constant_values=0)
    dst = jnp.pad(dst, ((0, 0), (0, CHUNK - E // 32)), constant_values=N)
    return src.reshape(32 * CHUNK // 128, 128), dst.reshape(32 * CHUNK // 128, 128)


def kernel(x_s, x_t, edge_index_s, edge_index_t, xs_batch, xt_batch,
           W_s1, a_src_s1, a_dst_s1, b_s1, W_s2, a_src_s2, a_dst_s2, b_s2,
           W_t1, a_src_t1, a_dst_t1, b_t1, W_t2, a_src_t2, a_dst_t2, b_t2,
           W_lin, b_lin):
    src_s, dst_s = _pad_edges(edge_index_s)
    src_t, dst_t = _pad_edges(edge_index_t)
    xsb3 = xs_batch.reshape(RB, 1, RBS)
    xtb3 = xt_batch.reshape(RB, 1, RBS)

    h1, as1, ad1 = _tc_head1(x_s, W_s1, a_src_s1, a_dst_s1, 64)
    op1 = _sc_gat_edges(h1, as1, ad1, src_s, dst_s, 64)
    h2, as2, ad2 = _tc_head2(op1, b_s1, W_s2, a_src_s2, a_dst_s2, 32)
    op2 = _sc_gat_edges(h2, as2, ad2, src_s, dst_s, 32)

    h3, as3, ad3 = _tc_head1(x_t, W_t1, a_src_t1, a_dst_t1, 64)
    op3 = _sc_gat_edges(h3, as3, ad3, src_t, dst_t, 64)
    h4, as4, ad4 = _tc_head2(op3, b_t1, W_t2, a_src_t2, a_dst_t2, 32)
    op4 = _sc_gat_edges(h4, as4, ad4, src_t, dst_t, 32)

    return _tc_pool_final(op2, b_s2, op4, b_t2, xsb3, xtb3, W_lin, b_lin)


# speedup vs baseline: 1.0219x; 1.0219x over previous
import functools

import jax
import jax.numpy as jnp
from jax import lax
from jax.experimental import pallas as pl
from jax.experimental.pallas import tpu as pltpu
from jax.experimental.pallas import tpu_sc as plsc

N = 10000
E = 320000
B = 64
NP = N + 16
CHUNK = 10240
RB = 10
RBS = N // RB


def _tc_head1(x, W, a_s, a_d, dout):
    def body(x_ref, w_ref, as_ref, ad_ref, h_ref, aso_ref, ado_ref):
        h = jnp.dot(x_ref[...], w_ref[...], preferred_element_type=jnp.float32)
        h_ref[...] = h.astype(jnp.bfloat16)
        asv = lax.dot_general(as_ref[...], h, (((1,), (1,)), ((), ())),
                              preferred_element_type=jnp.float32)
        adv = lax.dot_general(ad_ref[...], h, (((1,), (1,)), ((), ())),
                              preferred_element_type=jnp.float32)
        aso_ref[...] = asv.reshape(1, 1, RBS)
        ado_ref[...] = adv.reshape(1, 1, RBS)

    din = x.shape[1]
    h, as3, ad3 = pl.pallas_call(
        body,
        grid=(RB,),
        in_specs=[
            pl.BlockSpec((RBS, din), lambda i: (i, 0)),
            pl.BlockSpec((din, dout), lambda i: (0, 0)),
            pl.BlockSpec((1, dout), lambda i: (0, 0)),
            pl.BlockSpec((1, dout), lambda i: (0, 0)),
        ],
        out_specs=[
            pl.BlockSpec((RBS, dout), lambda i: (i, 0)),
            pl.BlockSpec((1, 1, RBS), lambda i: (i, 0, 0)),
            pl.BlockSpec((1, 1, RBS), lambda i: (i, 0, 0)),
        ],
        out_shape=[
            jax.ShapeDtypeStruct((N, dout), jnp.bfloat16),
            jax.ShapeDtypeStruct((RB, 1, RBS), jnp.float32),
            jax.ShapeDtypeStruct((RB, 1, RBS), jnp.float32),
        ],
    )(x, W, a_s.reshape(1, dout), a_d.reshape(1, dout))
    return h, as3.reshape(N), ad3.reshape(N)


def _tc_head2(outp, b, W, a_s, a_d, dout):
    def body(op_ref, b_ref, w_ref, as_ref, ad_ref, h_ref, aso_ref, ado_ref):
        x = jax.nn.relu(op_ref[0].astype(jnp.float32) +
                        op_ref[1].astype(jnp.float32) + b_ref[...])
        h = jnp.dot(x, w_ref[...], preferred_element_type=jnp.float32)
        h_ref[...] = h.astype(jnp.bfloat16)
        asv = lax.dot_general(as_ref[...], h, (((1,), (1,)), ((), ())),
                              preferred_element_type=jnp.float32)
        adv = lax.dot_general(ad_ref[...], h, (((1,), (1,)), ((), ())),
                              preferred_element_type=jnp.float32)
        aso_ref[...] = asv.reshape(1, 1, RBS)
        ado_ref[...] = adv.reshape(1, 1, RBS)

    din = outp.shape[2]
    h, as3, ad3 = pl.pallas_call(
        body,
        grid=(RB,),
        in_specs=[
            pl.BlockSpec((2, RBS, din), lambda i: (0, i, 0)),
            pl.BlockSpec((1, din), lambda i: (0, 0)),
            pl.BlockSpec((din, dout), lambda i: (0, 0)),
            pl.BlockSpec((1, dout), lambda i: (0, 0)),
            pl.BlockSpec((1, dout), lambda i: (0, 0)),
        ],
        out_specs=[
            pl.BlockSpec((RBS, dout), lambda i: (i, 0)),
            pl.BlockSpec((1, 1, RBS), lambda i: (i, 0, 0)),
            pl.BlockSpec((1, 1, RBS), lambda i: (i, 0, 0)),
        ],
        out_shape=[
            jax.ShapeDtypeStruct((N, dout), jnp.bfloat16),
            jax.ShapeDtypeStruct((RB, 1, RBS), jnp.float32),
            jax.ShapeDtypeStruct((RB, 1, RBS), jnp.float32),
        ],
    )(outp, b.reshape(1, din), W, a_s.reshape(1, dout), a_d.reshape(1, dout))
    return h, as3.reshape(N), ad3.reshape(N)


_SC_PARAMS = pltpu.CompilerParams(needs_layout_passes=False,
                                  use_tc_tiling_on_sc=False)
_MESH = plsc.VectorSubcoreMesh(core_axis_name="c", subcore_axis_name="s")


def _sc_gat_den(asrc, adst, src2d, dst2d):
    @functools.partial(
        pl.kernel,
        out_type=[
            jax.ShapeDtypeStruct((32 * CHUNK // 128, 128), jnp.float32),
            jax.ShapeDtypeStruct((2, NP), jnp.float32),
        ],
        mesh=_MESH,
        compiler_params=_SC_PARAMS,
        scratch_types=[
            pltpu.VMEM((NP,), jnp.float32),
            pltpu.VMEM((NP,), jnp.float32),
            pltpu.VMEM((80, 128), jnp.int32),
            pltpu.VMEM((80, 128), jnp.int32),
            pltpu.VMEM((80, 128), jnp.float32),
            pltpu.VMEM((640,), jnp.float32),
            pltpu.VMEM_SHARED((NP,), jnp.float32),
            pltpu.SemaphoreType.DMA,
            pltpu.SemaphoreType.DMA,
        ],
    )
    def k(asrc_hbm, adst_hbm, src_hbm, dst_hbm, ex_hbm, den_hbm,
          asrc_loc, adst_loc, src_loc, dst_loc, ex_loc, zeros, den_sp,
          lsem, dsem):
        cid = lax.axis_index("c")
        sid = lax.axis_index("s")
        chunk = cid * 16 + sid
        z16 = jnp.zeros((16,), jnp.float32)

        pltpu.async_copy(asrc_hbm, asrc_loc.at[pl.ds(0, N)], lsem)
        pltpu.async_copy(adst_hbm, adst_loc.at[pl.ds(0, N)], lsem)
        pltpu.async_copy(src_hbm.at[pl.ds(chunk * 80, 80)], src_loc, lsem)
        pltpu.async_copy(dst_hbm.at[pl.ds(chunk * 80, 80)], dst_loc, lsem)

        @pl.loop(0, 640, step=16)
        def _(i):
            zeros[pl.ds(i, 16)] = z16

        @pl.when(sid == 0)
        def _():
            for t in range(15):
                pltpu.sync_copy(zeros, den_sp.at[pl.ds(t * 640, 640)])
            pltpu.sync_copy(zeros.at[pl.ds(0, 416)], den_sp.at[pl.ds(9600, 416)])

        pltpu.make_async_copy(asrc_hbm, asrc_loc.at[pl.ds(0, N)], lsem).wait()
        pltpu.make_async_copy(adst_hbm, adst_loc.at[pl.ds(0, N)], lsem).wait()
        pltpu.make_async_copy(src_hbm.at[pl.ds(chunk * 80, 80)], src_loc, lsem).wait()
        pltpu.make_async_copy(dst_hbm.at[pl.ds(chunk * 80, 80)], dst_loc, lsem).wait()
        asrc_loc[pl.ds(N, 16)] = z16
        adst_loc[pl.ds(N, 16)] = z16
        plsc.subcore_barrier()

        @pl.loop(0, 80)
        def _(jb):
            for g in range(8):
                sv = src_loc[jb, pl.ds(16 * g, 16)]
                dv = dst_loc[jb, pl.ds(16 * g, 16)]
                e = plsc.load_gather(asrc_loc, [sv]) + plsc.load_gather(adst_loc, [dv])
                e = jnp.where(e > 0, e, 0.2 * e)
                ex_loc[jb, pl.ds(16 * g, 16)] = jnp.exp(e)

        pltpu.async_copy(ex_loc, ex_hbm.at[pl.ds(chunk * 80, 80)], lsem)

        for b in range(5):
            @pl.loop(16 * b, 16 * (b + 1))
            def _(jb):
                pltpu.async_copy(ex_loc.at[jb], den_sp.at[dst_loc.at[jb]],
                                 dsem, add=True)

            @pl.loop(16 * b, 16 * (b + 1))
            def _(jb):
                pltpu.make_async_copy(ex_loc.at[jb], den_sp.at[dst_loc.at[jb]],
                                      dsem).wait()

        pltpu.make_async_copy(ex_loc, ex_hbm.at[pl.ds(chunk * 80, 80)], lsem).wait()
        plsc.subcore_barrier()

        @pl.when(sid == 0)
        def _():
            pltpu.sync_copy(den_sp, den_hbm.at[cid])

    return k(asrc, adst, src2d, dst2d)


def _sc_gat_agg(h, ex, den, src2d, dst2d, dout):
    P = 4 if dout <= 32 else 2

    @functools.partial(
        pl.kernel,
        out_type=jax.ShapeDtypeStruct((2, N, dout), jnp.bfloat16),
        mesh=_MESH,
        compiler_params=_SC_PARAMS,
        scratch_types=(
            [
                pltpu.VMEM((NP,), jnp.float32),
                pltpu.VMEM((NP,), jnp.float32),
                pltpu.VMEM((80, 128), jnp.int32),
                pltpu.VMEM((80, 128), jnp.int32),
                pltpu.VMEM((80, 128), jnp.float32),
            ]
            + [pltpu.VMEM((256, dout), jnp.bfloat16)] * (2 * P)
            + [
                pltpu.VMEM((256,), jnp.float32),
                pltpu.VMEM_SHARED((NP, dout), jnp.bfloat16),
            ]
            + [pltpu.SemaphoreType.DMA] * (1 + 2 * P)
        ),
    )
    def k(h_hbm, ex_hbm, den_hbm, src_hbm, dst_hbm, out_hbm, *s):
        den_loc, den1, src_loc, dst_loc, ex_loc = s[0:5]
        bufs = list(s[5:5 + P])
        sbs = list(s[5 + P:5 + 2 * P])
        alpha = s[5 + 2 * P]
        out_sp = s[6 + 2 * P]
        lsem = s[7 + 2 * P]
        gss = list(s[8 + 2 * P:8 + 3 * P])
        sss = list(s[8 + 3 * P:8 + 4 * P])
        sb0 = sbs[0]
        cid = lax.axis_index("c")
        sid = lax.axis_index("s")
        chunk = cid * 16 + sid
        z32b = jnp.zeros((32,), jnp.bfloat16)

        pltpu.async_copy(den_hbm.at[0], den_loc, lsem)
        pltpu.async_copy(den_hbm.at[1], den1, lsem)
        pltpu.async_copy(src_hbm.at[pl.ds(chunk * 80, 80)], src_loc, lsem)
        pltpu.async_copy(dst_hbm.at[pl.ds(chunk * 80, 80)], dst_loc, lsem)
        pltpu.async_copy(ex_hbm.at[pl.ds(chunk * 80, 80)], ex_loc, lsem)

        @pl.loop(0, 256)
        def _(r):
            for q in range(dout // 32):
                sb0[r, pl.ds(32 * q, 32)] = z32b

        zbase = sid * 626
        for t in range(2):
            pltpu.sync_copy(sb0, out_sp.at[pl.ds(zbase + 256 * t, 256)])
        pltpu.sync_copy(sb0.at[pl.ds(0, 114)], out_sp.at[pl.ds(zbase + 512, 114)])

        pltpu.make_async_copy(den_hbm.at[0], den_loc, lsem).wait()
        pltpu.make_async_copy(den_hbm.at[1], den1, lsem).wait()
        pltpu.make_async_copy(src_hbm.at[pl.ds(chunk * 80, 80)], src_loc, lsem).wait()
        pltpu.make_async_copy(dst_hbm.at[pl.ds(chunk * 80, 80)], dst_loc, lsem).wait()
        pltpu.make_async_copy(ex_hbm.at[pl.ds(chunk * 80, 80)], ex_loc, lsem).wait()

        @pl.loop(0, NP, step=16)
        def _(i):
            den_loc[pl.ds(i, 16)] = den_loc[pl.ds(i, 16)] + den1[pl.ds(i, 16)]

        plsc.subcore_barrier()

        def fire_gather(j, buf, t, sem):
            pltpu.async_copy(h_hbm.at[src_loc.at[j]],
                             buf.at[pl.ds(128 * t, 128)], sem)

        def wait_gather(j, buf, t, sem):
            pltpu.make_async_copy(h_hbm.at[src_loc.at[j]],
                                  buf.at[pl.ds(128 * t, 128)], sem).wait()

        def fire_scatter(j, sbuf, t, sem):
            pltpu.async_copy(sbuf.at[pl.ds(128 * t, 128)],
                             out_sp.at[dst_loc.at[j]], sem, add=True)

        def wait_scatter(j, sbuf, t, sem):
            pltpu.make_async_copy(sbuf.at[pl.ds(128 * t, 128)],
                                  out_sp.at[dst_loc.at[j]], sem).wait()

        def alpha_scale(jj, buf, sbuf):
            for t in range(2):
                for g in range(8):
                    dv = dst_loc[jj + t, pl.ds(16 * g, 16)]
                    dn = plsc.load_gather(den_loc, [dv])
                    exv = ex_loc[jj + t, pl.ds(16 * g, 16)]
                    alpha[pl.ds(128 * t + 16 * g, 16)] = exv / (dn + 1e-16)

            @pl.loop(0, 256, step=16)
            def _(e0):
                av16 = alpha[pl.ds(e0, 16)]
                for k2 in range(16):
                    avf = jnp.broadcast_to(av16[k2], (16,))
                    av = plsc.pack(avf, avf, format=plsc.PackFormat.INTERLEAVED)
                    for q in range(dout // 32):
                        sbuf[e0 + k2, pl.ds(32 * q, 32)] = (
                            buf[e0 + k2, pl.ds(32 * q, 32)] * av)

        S = 2 * P
        for p in range(P):
            fire_gather(2 * p, bufs[p], 0, gss[p])
            fire_gather(2 * p + 1, bufs[p], 1, gss[p])

        @pl.loop(0, 80, step=S)
        def _(jj):
            for p in range(P):
                b0 = jj + 2 * p
                wait_gather(b0, bufs[p], 0, gss[p])
                wait_gather(b0 + 1, bufs[p], 1, gss[p])

                @pl.when(jj > 0)
                def _():
                    wait_scatter(b0 - S, sbs[p], 0, sss[p])
                    wait_scatter(b0 - S + 1, sbs[p], 1, sss[p])

                alpha_scale(b0, bufs[p], sbs[p])
                fire_scatter(b0, sbs[p], 0, sss[p])
                fire_scatter(b0 + 1, sbs[p], 1, sss[p])

                @pl.when(jj < 80 - S)
                def _():
                    fire_gather(b0 + S, bufs[p], 0, gss[p])
                    fire_gather(b0 + S + 1, bufs[p], 1, gss[p])

        for p in range(P):
            wait_scatter(80 - S + 2 * p, sbs[p], 0, sss[p])
            wait_scatter(80 - S + 2 * p + 1, sbs[p], 1, sss[p])
        plsc.subcore_barrier()
        wb = sid * 624
        pltpu.sync_copy(out_sp.at[pl.ds(wb, 624)], out_hbm.at[cid, pl.ds(wb, 624)])

        @pl.when(sid == 15)
        def _():
            pltpu.sync_copy(out_sp.at[pl.ds(9984, 16)],
                            out_hbm.at[cid, pl.ds(9984, 16)])

    return k(h, ex, den, src2d, dst2d)


def _sc_gat_edges(h, asrc, adst, src2d, dst2d, dout):
    ex, den = _sc_gat_den(asrc, adst, src2d, dst2d)
    return _sc_gat_agg(h, ex, den, src2d, dst2d, dout)


def _tc_pool_final(op_s, b_s, op_t, b_t, xsb3, xtb3, W_lin, b_lin):
    def body(ops_ref, bs_ref, opt_ref, bt_ref, xsb_ref, xtb_ref, wl_ref, bl_ref,
             out_ref, accs, cnts, acct, cntt):
        i = pl.program_id(0)

        @pl.when(i == 0)
        def _():
            accs[...] = jnp.zeros_like(accs)
            cnts[...] = jnp.zeros_like(cnts)
            acct[...] = jnp.zeros_like(acct)
            cntt[...] = jnp.zeros_like(cntt)

        iot = lax.broadcasted_iota(jnp.int32, (B, RBS), 0)
        x2s = jax.nn.relu(ops_ref[0].astype(jnp.float32) +
                          ops_ref[1].astype(jnp.float32) + bs_ref[...])
        ms = (xsb_ref[0, 0, :][None, :] == iot).astype(jnp.float32)
        accs[...] += jnp.dot(ms, x2s, preferred_element_type=jnp.float32)
        cnts[...] += jnp.sum(ms, axis=1, keepdims=True)
        x2t = jax.nn.relu(opt_ref[0].astype(jnp.float32) +
                          opt_ref[1].astype(jnp.float32) + bt_ref[...])
        mt = (xtb_ref[0, 0, :][None, :] == iot).astype(jnp.float32)
        acct[...] += jnp.dot(mt, x2t, preferred_element_type=jnp.float32)
        cntt[...] += jnp.sum(mt, axis=1, keepdims=True)

        @pl.when(i == RB - 1)
        def _():
            xs = accs[...] / jnp.maximum(cnts[...], 1.0)
            xt = acct[...] / jnp.maximum(cntt[...], 1.0)
            o = jnp.dot(xs + xt, wl_ref[...], preferred_element_type=jnp.float32)
            out_ref[...] = jax.nn.sigmoid(o + bl_ref[...])

    din = op_s.shape[2]
    return pl.pallas_call(
        body,
        grid=(RB,),
        in_specs=[
            pl.BlockSpec((2, RBS, din), lambda i: (0, i, 0)),
            pl.BlockSpec((1, din), lambda i: (0, 0)),
            pl.BlockSpec((2, RBS, din), lambda i: (0, i, 0)),
            pl.BlockSpec((1, din), lambda i: (0, 0)),
            pl.BlockSpec((1, 1, RBS), lambda i: (i, 0, 0)),
            pl.BlockSpec((1, 1, RBS), lambda i: (i, 0, 0)),
            pl.BlockSpec((din, 1), lambda i: (0, 0)),
            pl.BlockSpec((1, 1), lambda i: (0, 0)),
        ],
        out_specs=pl.BlockSpec((B, 1), lambda i: (0, 0)),
        out_shape=jax.ShapeDtypeStruct((B, 1), jnp.float32),
        scratch_shapes=[
            pltpu.VMEM((B, din), jnp.float32),
            pltpu.VMEM((B, 1), jnp.float32),
            pltpu.VMEM((B, din), jnp.float32),
            pltpu.VMEM((B, 1), jnp.float32),
        ],
    )(op_s, b_s.reshape(1, din), op_t, b_t.reshape(1, din),
      xsb3, xtb3, W_lin, b_lin.reshape(1, 1))


def _pad_edges(edge_index):
    src = edge_index[0].reshape(32, E // 32)
    dst = edge_index[1].reshape(32, E // 32)
    src = jnp.pad(src, ((0, 0), (0, CHUNK - E // 32)), constant_values=0)
    dst = jnp.pad(dst, ((0, 0), (0, CHUNK - E // 32)), constant_values=N)
    return src.reshape(32 * CHUNK // 128, 128), dst.reshape(32 * CHUNK // 128, 128)


def kernel(x_s, x_t, edge_index_s, edge_index_t, xs_batch, xt_batch,
           W_s1, a_src_s1, a_dst_s1, b_s1, W_s2, a_src_s2, a_dst_s2, b_s2,
           W_t1, a_src_t1, a_dst_t1, b_t1, W_t2, a_src_t2, a_dst_t2, b_t2,
           W_lin, b_lin):
    src_s, dst_s = _pad_edges(edge_index_s)
    src_t, dst_t = _pad_edges(edge_index_t)
    xsb3 = xs_batch.reshape(RB, 1, RBS)
    xtb3 = xt_batch.reshape(RB, 1, RBS)

    h1, as1, ad1 = _tc_head1(x_s, W_s1, a_src_s1, a_dst_s1, 64)
    op1 = _sc_gat_edges(h1, as1, ad1, src_s, dst_s, 64)
    h2, as2, ad2 = _tc_head2(op1, b_s1, W_s2, a_src_s2, a_dst_s2, 32)
    op2 = _sc_gat_edges(h2, as2, ad2, src_s, dst_s, 32)

    h3, as3, ad3 = _tc_head1(x_t, W_t1, a_src_t1, a_dst_t1, 64)
    op3 = _sc_gat_edges(h3, as3, ad3, src_t, dst_t, 64)
    h4, as4, ad4 = _tc_head2(op3, b_t1, W_t2, a_src_t2, a_dst_t2, 32)
    op4 = _sc_gat_edges(h4, as4, ad4, src_t, dst_t, 32)

    return _tc_pool_final(op2, b_s2, op4, b_t2, xsb3, xtb3, W_lin, b_lin)


# speedup vs baseline: 1.0427x; 1.0204x over previous
import functools

import jax
import jax.numpy as jnp
from jax import lax
from jax.experimental import pallas as pl
from jax.experimental.pallas import tpu as pltpu
from jax.experimental.pallas import tpu_sc as plsc

N = 10000
E = 320000
B = 64
NP = N + 16
CHUNK = 10240
RB = 10
RBS = N // RB


def _tc_head1(x, W, a_s, a_d, dout):
    def body(x_ref, w_ref, as_ref, ad_ref, h_ref, aso_ref, ado_ref):
        h = jnp.dot(x_ref[...], w_ref[...], preferred_element_type=jnp.float32)
        h_ref[...] = h.astype(jnp.bfloat16)
        asv = lax.dot_general(as_ref[...], h, (((1,), (1,)), ((), ())),
                              preferred_element_type=jnp.float32)
        adv = lax.dot_general(ad_ref[...], h, (((1,), (1,)), ((), ())),
                              preferred_element_type=jnp.float32)
        aso_ref[...] = asv.reshape(1, 1, RBS)
        ado_ref[...] = adv.reshape(1, 1, RBS)

    din = x.shape[1]
    h, as3, ad3 = pl.pallas_call(
        body,
        grid=(RB,),
        in_specs=[
            pl.BlockSpec((RBS, din), lambda i: (i, 0)),
            pl.BlockSpec((din, dout), lambda i: (0, 0)),
            pl.BlockSpec((1, dout), lambda i: (0, 0)),
            pl.BlockSpec((1, dout), lambda i: (0, 0)),
        ],
        out_specs=[
            pl.BlockSpec((RBS, dout), lambda i: (i, 0)),
            pl.BlockSpec((1, 1, RBS), lambda i: (i, 0, 0)),
            pl.BlockSpec((1, 1, RBS), lambda i: (i, 0, 0)),
        ],
        out_shape=[
            jax.ShapeDtypeStruct((N, dout), jnp.bfloat16),
            jax.ShapeDtypeStruct((RB, 1, RBS), jnp.float32),
            jax.ShapeDtypeStruct((RB, 1, RBS), jnp.float32),
        ],
    )(x, W, a_s.reshape(1, dout), a_d.reshape(1, dout))
    return h, as3.reshape(N), ad3.reshape(N)


def _tc_head2(outp, b, W, a_s, a_d, dout):
    def body(op_ref, b_ref, w_ref, as_ref, ad_ref, h_ref, aso_ref, ado_ref):
        x = jax.nn.relu(op_ref[0].astype(jnp.float32) +
                        op_ref[1].astype(jnp.float32) + b_ref[...])
        h = jnp.dot(x, w_ref[...], preferred_element_type=jnp.float32)
        h_ref[...] = h.astype(jnp.bfloat16)
        asv = lax.dot_general(as_ref[...], h, (((1,), (1,)), ((), ())),
                              preferred_element_type=jnp.float32)
        adv = lax.dot_general(ad_ref[...], h, (((1,), (1,)), ((), ())),
                              preferred_element_type=jnp.float32)
        aso_ref[...] = asv.reshape(1, 1, RBS)
        ado_ref[...] = adv.reshape(1, 1, RBS)

    din = outp.shape[2]
    h, as3, ad3 = pl.pallas_call(
        body,
        grid=(RB,),
        in_specs=[
            pl.BlockSpec((2, RBS, din), lambda i: (0, i, 0)),
            pl.BlockSpec((1, din), lambda i: (0, 0)),
            pl.BlockSpec((din, dout), lambda i: (0, 0)),
            pl.BlockSpec((1, dout), lambda i: (0, 0)),
            pl.BlockSpec((1, dout), lambda i: (0, 0)),
        ],
        out_specs=[
            pl.BlockSpec((RBS, dout), lambda i: (i, 0)),
            pl.BlockSpec((1, 1, RBS), lambda i: (i, 0, 0)),
            pl.BlockSpec((1, 1, RBS), lambda i: (i, 0, 0)),
        ],
        out_shape=[
            jax.ShapeDtypeStruct((N, dout), jnp.bfloat16),
            jax.ShapeDtypeStruct((RB, 1, RBS), jnp.float32),
            jax.ShapeDtypeStruct((RB, 1, RBS), jnp.float32),
        ],
    )(outp, b.reshape(1, din), W, a_s.reshape(1, dout), a_d.reshape(1, dout))
    return h, as3.reshape(N), ad3.reshape(N)


_SC_PARAMS = pltpu.CompilerParams(needs_layout_passes=False,
                                  use_tc_tiling_on_sc=False)
_MESH = plsc.VectorSubcoreMesh(core_axis_name="c", subcore_axis_name="s")


def _sc_gat_den(asrc, adst, src2d, dst2d):
    @functools.partial(
        pl.kernel,
        out_type=[
            jax.ShapeDtypeStruct((32 * CHUNK // 128, 128), jnp.float32),
            jax.ShapeDtypeStruct((2, NP), jnp.float32),
        ],
        mesh=_MESH,
        compiler_params=_SC_PARAMS,
        scratch_types=[
            pltpu.VMEM((NP,), jnp.float32),
            pltpu.VMEM((NP,), jnp.float32),
            pltpu.VMEM((80, 128), jnp.int32),
            pltpu.VMEM((80, 128), jnp.int32),
            pltpu.VMEM((80, 128), jnp.float32),
            pltpu.VMEM((640,), jnp.float32),
            pltpu.VMEM_SHARED((NP,), jnp.float32),
            pltpu.SemaphoreType.DMA,
            pltpu.SemaphoreType.DMA,
        ],
    )
    def k(asrc_hbm, adst_hbm, src_hbm, dst_hbm, ex_hbm, den_hbm,
          asrc_loc, adst_loc, src_loc, dst_loc, ex_loc, zeros, den_sp,
          lsem, dsem):
        cid = lax.axis_index("c")
        sid = lax.axis_index("s")
        chunk = cid * 16 + sid
        z16 = jnp.zeros((16,), jnp.float32)

        pltpu.async_copy(asrc_hbm, asrc_loc.at[pl.ds(0, N)], lsem)
        pltpu.async_copy(adst_hbm, adst_loc.at[pl.ds(0, N)], lsem)
        pltpu.async_copy(src_hbm.at[pl.ds(chunk * 80, 80)], src_loc, lsem)
        pltpu.async_copy(dst_hbm.at[pl.ds(chunk * 80, 80)], dst_loc, lsem)

        @pl.loop(0, 640, step=16)
        def _(i):
            zeros[pl.ds(i, 16)] = z16

        @pl.when(sid == 0)
        def _():
            for t in range(15):
                pltpu.sync_copy(zeros, den_sp.at[pl.ds(t * 640, 640)])
            pltpu.sync_copy(zeros.at[pl.ds(0, 416)], den_sp.at[pl.ds(9600, 416)])

        pltpu.make_async_copy(asrc_hbm, asrc_loc.at[pl.ds(0, N)], lsem).wait()
        pltpu.make_async_copy(adst_hbm, adst_loc.at[pl.ds(0, N)], lsem).wait()
        pltpu.make_async_copy(src_hbm.at[pl.ds(chunk * 80, 80)], src_loc, lsem).wait()
        pltpu.make_async_copy(dst_hbm.at[pl.ds(chunk * 80, 80)], dst_loc, lsem).wait()
        asrc_loc[pl.ds(N, 16)] = z16
        adst_loc[pl.ds(N, 16)] = z16
        plsc.subcore_barrier()

        @pl.loop(0, 80)
        def _(jb):
            for g in range(8):
                sv = src_loc[jb, pl.ds(16 * g, 16)]
                dv = dst_loc[jb, pl.ds(16 * g, 16)]
                e = plsc.load_gather(asrc_loc, [sv]) + plsc.load_gather(adst_loc, [dv])
                e = jnp.where(e > 0, e, 0.2 * e)
                ex_loc[jb, pl.ds(16 * g, 16)] = jnp.exp(e)

        pltpu.async_copy(ex_loc, ex_hbm.at[pl.ds(chunk * 80, 80)], lsem)

        for b in range(5):
            @pl.loop(16 * b, 16 * (b + 1))
            def _(jb):
                pltpu.async_copy(ex_loc.at[jb], den_sp.at[dst_loc.at[jb]],
                                 dsem, add=True)

            @pl.loop(16 * b, 16 * (b + 1))
            def _(jb):
                pltpu.make_async_copy(ex_loc.at[jb], den_sp.at[dst_loc.at[jb]],
                                      dsem).wait()

        pltpu.make_async_copy(ex_loc, ex_hbm.at[pl.ds(chunk * 80, 80)], lsem).wait()
        plsc.subcore_barrier()

        @pl.when(sid == 0)
        def _():
            pltpu.sync_copy(den_sp, den_hbm.at[cid])

    return k(asrc, adst, src2d, dst2d)


def _sc_gat_agg(h, ex, den, src2d, dst2d, dout):
    P = 8 if dout <= 32 else 5

    @functools.partial(
        pl.kernel,
        out_type=jax.ShapeDtypeStruct((2, N, dout), jnp.bfloat16),
        mesh=_MESH,
        compiler_params=_SC_PARAMS,
        scratch_types=(
            [
                pltpu.VMEM((NP,), jnp.float32),
                pltpu.VMEM((NP,), jnp.float32),
                pltpu.VMEM((80, 128), jnp.int32),
                pltpu.VMEM((80, 128), jnp.int32),
                pltpu.VMEM((80, 128), jnp.float32),
            ]
            + [pltpu.VMEM((128, dout), jnp.bfloat16)] * (2 * P)
            + [
                pltpu.VMEM((128,), jnp.float32),
                pltpu.VMEM_SHARED((NP, dout), jnp.bfloat16),
            ]
            + [pltpu.SemaphoreType.DMA] * (1 + 2 * P)
        ),
    )
    def k(h_hbm, ex_hbm, den_hbm, src_hbm, dst_hbm, out_hbm, *s):
        den_loc, den1, src_loc, dst_loc, ex_loc = s[0:5]
        bufs = list(s[5:5 + P])
        sbs = list(s[5 + P:5 + 2 * P])
        alpha = s[5 + 2 * P]
        out_sp = s[6 + 2 * P]
        lsem = s[7 + 2 * P]
        gss = list(s[8 + 2 * P:8 + 3 * P])
        sss = list(s[8 + 3 * P:8 + 4 * P])
        sb0 = sbs[0]
        cid = lax.axis_index("c")
        sid = lax.axis_index("s")
        chunk = cid * 16 + sid
        z32b = jnp.zeros((32,), jnp.bfloat16)

        pltpu.async_copy(den_hbm.at[0], den_loc, lsem)
        pltpu.async_copy(den_hbm.at[1], den1, lsem)
        pltpu.async_copy(src_hbm.at[pl.ds(chunk * 80, 80)], src_loc, lsem)
        pltpu.async_copy(dst_hbm.at[pl.ds(chunk * 80, 80)], dst_loc, lsem)
        pltpu.async_copy(ex_hbm.at[pl.ds(chunk * 80, 80)], ex_loc, lsem)

        @pl.loop(0, 128)
        def _(r):
            for q in range(dout // 32):
                sb0[r, pl.ds(32 * q, 32)] = z32b

        zbase = sid * 626
        for t in range(4):
            pltpu.sync_copy(sb0, out_sp.at[pl.ds(zbase + 128 * t, 128)])
        pltpu.sync_copy(sb0.at[pl.ds(0, 114)], out_sp.at[pl.ds(zbase + 512, 114)])

        pltpu.make_async_copy(den_hbm.at[0], den_loc, lsem).wait()
        pltpu.make_async_copy(den_hbm.at[1], den1, lsem).wait()
        pltpu.make_async_copy(src_hbm.at[pl.ds(chunk * 80, 80)], src_loc, lsem).wait()
        pltpu.make_async_copy(dst_hbm.at[pl.ds(chunk * 80, 80)], dst_loc, lsem).wait()
        pltpu.make_async_copy(ex_hbm.at[pl.ds(chunk * 80, 80)], ex_loc, lsem).wait()

        @pl.loop(0, NP, step=16)
        def _(i):
            den_loc[pl.ds(i, 16)] = den_loc[pl.ds(i, 16)] + den1[pl.ds(i, 16)]

        plsc.subcore_barrier()

        def fire_gather(j, buf, sem):
            pltpu.async_copy(h_hbm.at[src_loc.at[j]], buf, sem)

        def wait_gather(j, buf, sem):
            pltpu.make_async_copy(h_hbm.at[src_loc.at[j]], buf, sem).wait()

        def fire_scatter(j, sbuf, sem):
            pltpu.async_copy(sbuf, out_sp.at[dst_loc.at[j]], sem, add=True)

        def wait_scatter(j, sbuf, sem):
            pltpu.make_async_copy(sbuf, out_sp.at[dst_loc.at[j]], sem).wait()

        def alpha_scale(jb, buf, sbuf):
            for g in range(8):
                dv = dst_loc[jb, pl.ds(16 * g, 16)]
                dn = plsc.load_gather(den_loc, [dv])
                exv = ex_loc[jb, pl.ds(16 * g, 16)]
                alpha[pl.ds(16 * g, 16)] = exv / (dn + 1e-16)

            @pl.loop(0, 128, step=16)
            def _(e0):
                av16 = alpha[pl.ds(e0, 16)]
                for k2 in range(16):
                    avf = jnp.broadcast_to(av16[k2], (16,))
                    av = plsc.pack(avf, avf, format=plsc.PackFormat.INTERLEAVED)
                    for q in range(dout // 32):
                        sbuf[e0 + k2, pl.ds(32 * q, 32)] = (
                            buf[e0 + k2, pl.ds(32 * q, 32)] * av)

        for p in range(P):
            fire_gather(p, bufs[p], gss[p])

        @pl.loop(0, 80, step=P)
        def _(jj):
            for p in range(P):
                b0 = jj + p
                wait_gather(b0, bufs[p], gss[p])

                @pl.when(jj > 0)
                def _():
                    wait_scatter(b0 - P, sbs[p], sss[p])

                alpha_scale(b0, bufs[p], sbs[p])
                fire_scatter(b0, sbs[p], sss[p])

                @pl.when(jj < 80 - P)
                def _():
                    fire_gather(b0 + P, bufs[p], gss[p])

        for p in range(P):
            wait_scatter(80 - P + p, sbs[p], sss[p])
        plsc.subcore_barrier()
        wb = sid * 624
        pltpu.sync_copy(out_sp.at[pl.ds(wb, 624)], out_hbm.at[cid, pl.ds(wb, 624)])

        @pl.when(sid == 15)
        def _():
            pltpu.sync_copy(out_sp.at[pl.ds(9984, 16)],
                            out_hbm.at[cid, pl.ds(9984, 16)])

    return k(h, ex, den, src2d, dst2d)


def _sc_gat_edges(h, asrc, adst, src2d, dst2d, dout):
    ex, den = _sc_gat_den(asrc, adst, src2d, dst2d)
    return _sc_gat_agg(h, ex, den, src2d, dst2d, dout)


def _tc_pool_final(op_s, b_s, op_t, b_t, xsb3, xtb3, W_lin, b_lin):
    def body(ops_ref, bs_ref, opt_ref, bt_ref, xsb_ref, xtb_ref, wl_ref, bl_ref,
             out_ref, accs, cnts, acct, cntt):
        i = pl.program_id(0)

        @pl.when(i == 0)
        def _():
            accs[...] = jnp.zeros_like(accs)
            cnts[...] = jnp.zeros_like(cnts)
            acct[...] = jnp.zeros_like(acct)
            cntt[...] = jnp.zeros_like(cntt)

        iot = lax.broadcasted_iota(jnp.int32, (B, RBS), 0)
        x2s = jax.nn.relu(ops_ref[0].astype(jnp.float32) +
                          ops_ref[1].astype(jnp.float32) + bs_ref[...])
        ms = (xsb_ref[0, 0, :][None, :] == iot).astype(jnp.float32)
        accs[...] += jnp.dot(ms, x2s, preferred_element_type=jnp.float32)
        cnts[...] += jnp.sum(ms, axis=1, keepdims=True)
        x2t = jax.nn.relu(opt_ref[0].astype(jnp.float32) +
                          opt_ref[1].astype(jnp.float32) + bt_ref[...])
        mt = (xtb_ref[0, 0, :][None, :] == iot).astype(jnp.float32)
        acct[...] += jnp.dot(mt, x2t, preferred_element_type=jnp.float32)
        cntt[...] += jnp.sum(mt, axis=1, keepdims=True)

        @pl.when(i == RB - 1)
        def _():
            xs = accs[...] / jnp.maximum(cnts[...], 1.0)
            xt = acct[...] / jnp.maximum(cntt[...], 1.0)
            o = jnp.dot(xs + xt, wl_ref[...], preferred_element_type=jnp.float32)
            out_ref[...] = jax.nn.sigmoid(o + bl_ref[...])

    din = op_s.shape[2]
    return pl.pallas_call(
        body,
        grid=(RB,),
        in_specs=[
            pl.BlockSpec((2, RBS, din), lambda i: (0, i, 0)),
            pl.BlockSpec((1, din), lambda i: (0, 0)),
            pl.BlockSpec((2, RBS, din), lambda i: (0, i, 0)),
            pl.BlockSpec((1, din), lambda i: (0, 0)),
            pl.BlockSpec((1, 1, RBS), lambda i: (i, 0, 0)),
            pl.BlockSpec((1, 1, RBS), lambda i: (i, 0, 0)),
            pl.BlockSpec((din, 1), lambda i: (0, 0)),
            pl.BlockSpec((1, 1), lambda i: (0, 0)),
        ],
        out_specs=pl.BlockSpec((B, 1), lambda i: (0, 0)),
        out_shape=jax.ShapeDtypeStruct((B, 1), jnp.float32),
        scratch_shapes=[
            pltpu.VMEM((B, din), jnp.float32),
            pltpu.VMEM((B, 1), jnp.float32),
            pltpu.VMEM((B, din), jnp.float32),
            pltpu.VMEM((B, 1), jnp.float32),
        ],
    )(op_s, b_s.reshape(1, din), op_t, b_t.reshape(1, din),
      xsb3, xtb3, W_lin, b_lin.reshape(1, 1))


def _pad_edges(edge_index):
    src = edge_index[0].reshape(32, E // 32)
    dst = edge_index[1].reshape(32, E // 32)
    src = jnp.pad(src, ((0, 0), (0, CHUNK - E // 32)), constant_values=0)
    dst = jnp.pad(dst, ((0, 0), (0, CHUNK - E // 32)), constant_values=N)
    return src.reshape(32 * CHUNK // 128, 128), dst.reshape(32 * CHUNK // 128, 128)


def kernel(x_s, x_t, edge_index_s, edge_index_t, xs_batch, xt_batch,
           W_s1, a_src_s1, a_dst_s1, b_s1, W_s2, a_src_s2, a_dst_s2, b_s2,
           W_t1, a_src_t1, a_dst_t1, b_t1, W_t2, a_src_t2, a_dst_t2, b_t2,
           W_lin, b_lin):
    src_s, dst_s = _pad_edges(edge_index_s)
    src_t, dst_t = _pad_edges(edge_index_t)
    xsb3 = xs_batch.reshape(RB, 1, RBS)
    xtb3 = xt_batch.reshape(RB, 1, RBS)

    h1, as1, ad1 = _tc_head1(x_s, W_s1, a_src_s1, a_dst_s1, 64)
    op1 = _sc_gat_edges(h1, as1, ad1, src_s, dst_s, 64)
    h2, as2, ad2 = _tc_head2(op1, b_s1, W_s2, a_src_s2, a_dst_s2, 32)
    op2 = _sc_gat_edges(h2, as2, ad2, src_s, dst_s, 32)

    h3, as3, ad3 = _tc_head1(x_t, W_t1, a_src_t1, a_dst_t1, 64)
    op3 = _sc_gat_edges(h3, as3, ad3, src_t, dst_t, 64)
    h4, as4, ad4 = _tc_head2(op3, b_t1, W_t2, a_src_t2, a_dst_t2, 32)
    op4 = _sc_gat_edges(h4, as4, ad4, src_t, dst_t, 32)

    return _tc_pool_final(op2, b_s2, op4, b_t2, xsb3, xtb3, W_lin, b_lin)
